# baseline (speedup 1.0000x reference)
import functools

import jax
import jax.numpy as jnp
from jax import lax
from jax.experimental import pallas as pl
from jax.experimental.pallas import tpu as pltpu
from jax.experimental.pallas import tpu_sc as plsc

_INFO = plsc.get_sparse_core_info()
_NC = _INFO.num_cores
_NS = _INFO.num_subcores
_NW = _NC * _NS


@functools.partial(jax.jit, static_argnums=(2,))
def _embedding_lookup(table, idx, nbuf):
    Bt, S = idx.shape
    V, D = table.shape
    seq_per_w = Bt // _NW
    mesh = plsc.VectorSubcoreMesh(core_axis_name="c", subcore_axis_name="s")

    @functools.partial(
        pl.kernel,
        out_type=jax.ShapeDtypeStruct((Bt, S, D), jnp.float32),
        mesh=mesh,
        compiler_params=pltpu.CompilerParams(use_tc_tiling_on_sc=False),
        scratch_types=[
            pltpu.VMEM((seq_per_w, S), jnp.int32),
            pltpu.VMEM((nbuf, S, D), jnp.float32),
            pltpu.SemaphoreType.DMA((nbuf,)),
            pltpu.SemaphoreType.DMA((nbuf,)),
        ],
    )
    def emb(table_hbm, idx_hbm, out_hbm, idx_v, rows_v, gsem, wsem):
        wid = lax.axis_index("s") * _NC + lax.axis_index("c")
        base = wid * seq_per_w

        def start_gather(i, b):
            pltpu.async_copy(table_hbm.at[idx_v.at[i]], rows_v.at[b],
                             gsem.at[b])

        def wait_gather(i, b):
            pltpu.make_async_copy(table_hbm.at[idx_v.at[i]], rows_v.at[b],
                                  gsem.at[b]).wait()

        def start_wb(i, b):
            pltpu.async_copy(rows_v.at[b], out_hbm.at[base + i], wsem.at[b])

        def wait_wb(i, b):
            pltpu.make_async_copy(rows_v.at[b], out_hbm.at[base + i],
                                  wsem.at[b]).wait()

        pltpu.sync_copy(idx_hbm.at[pl.ds(base, seq_per_w)], idx_v)

        for b in range(nbuf):
            start_gather(b, b)

        @pl.loop(0, seq_per_w - nbuf, step=nbuf)
        def ring(g):
            for b in range(nbuf):
                i = g + b
                wait_gather(i, b)
                start_wb(i, b)
                wait_wb(i, b)
                start_gather(i + nbuf, b)

        for b in range(nbuf):
            i = seq_per_w - nbuf + b
            wait_gather(i, b)
            start_wb(i, b)
        for b in range(nbuf):
            wait_wb(seq_per_w - nbuf + b, b)

    return emb(table, idx)


def kernel(token_indices, embedding_table):
    return _embedding_lookup(embedding_table,
                             token_indices.astype(jnp.int32), 4)
